# Initial kernel scaffold; baseline (speedup 1.0000x reference)
import jax
import jax.numpy as jnp
from jax.experimental import pallas as pl


def kernel(node_features, edge_indices, edge_attrs, W, b, gamma, beta, global_emb):
    raise NotImplementedError("write your pallas kernel here")


# speedup vs baseline: 30.2163x; 30.2163x over previous
import jax
import jax.numpy as jnp
from jax import lax
from jax.experimental import pallas as pl
from jax.experimental.pallas import tpu as pltpu

_L = 3


def _tgn_kernel(x_ref, src_ref, dst_ref, ea_ref, w_ref, b_ref, g_ref,
                be_ref, ge_ref, out_ref):
    x0 = x_ref[0]
    n = x0.shape[0]
    e = ea_ref.shape[1]
    ea = ea_ref[0]

    src_col = src_ref[0]
    dst_row = dst_ref[0]

    s_mat = (src_col == lax.broadcasted_iota(jnp.int32, (e, n), 1)
             ).astype(jnp.float32)
    p_mat = (lax.broadcasted_iota(jnp.int32, (n, e), 0) == dst_row
             ).astype(jnp.float32)

    x = x0
    for i in range(_L):
        residual = x
        h = jnp.dot(x, w_ref[i], preferred_element_type=jnp.float32)
        gathered = jnp.dot(s_mat, h, preferred_element_type=jnp.float32)
        msg = gathered * ea
        out = jnp.dot(p_mat, msg, preferred_element_type=jnp.float32)
        out = out + b_ref[i]
        x = jnp.maximum(out, 0.0)
        mu = jnp.mean(x, axis=-1, keepdims=True)
        var = jnp.mean((x - mu) * (x - mu), axis=-1, keepdims=True)
        x = (x - mu) * lax.rsqrt(var + 1e-5) * g_ref[i] + be_ref[i]
        if i > 0:
            x = x + residual

    presence = (jnp.sum(x0, axis=1, keepdims=True) != 0.0
                ).astype(jnp.float32)
    out_ref[0] = x * presence + ge_ref[...] * (1.0 - presence)


def kernel(node_features, edge_indices, edge_attrs, W, b, gamma, beta,
           global_emb):
    bsz, n, d = node_features.shape
    e = edge_attrs.shape[1]
    ei = edge_indices.astype(jnp.int32)
    src_col = ei[:, 0, :].reshape(bsz, e, 1)
    dst_row = ei[:, 1, :].reshape(bsz, 1, e)

    grid = (bsz,)
    return pl.pallas_call(
        _tgn_kernel,
        grid=grid,
        in_specs=[
            pl.BlockSpec((1, n, d), lambda g: (g, 0, 0)),
            pl.BlockSpec((1, e, 1), lambda g: (g, 0, 0)),
            pl.BlockSpec((1, 1, e), lambda g: (g, 0, 0)),
            pl.BlockSpec((1, e, d), lambda g: (g, 0, 0)),
            pl.BlockSpec((_L, d, d), lambda g: (0, 0, 0)),
            pl.BlockSpec((_L, d), lambda g: (0, 0)),
            pl.BlockSpec((_L, d), lambda g: (0, 0)),
            pl.BlockSpec((_L, d), lambda g: (0, 0)),
            pl.BlockSpec((n, d), lambda g: (0, 0)),
        ],
        out_specs=pl.BlockSpec((1, n, d), lambda g: (g, 0, 0)),
        out_shape=jax.ShapeDtypeStruct((bsz, n, d), jnp.float32),
    )(node_features, src_col, dst_row, edge_attrs, W, b, gamma, beta,
      global_emb)
